# baseline (speedup 1.0000x reference)
import jax
import jax.numpy as jnp
from jax.experimental import pallas as pl

_N = 1_000_000
_ROWS = 1000
_COLS = 1000


def _copy_body(in_ref, out_ref):
    out_ref[...] = in_ref[...]


def kernel(goal_logits):
    x = goal_logits.reshape(_ROWS, _COLS)
    out = pl.pallas_call(
        _copy_body,
        out_shape=jax.ShapeDtypeStruct((_ROWS, _COLS), jnp.float32),
        grid=(5,),
        in_specs=[pl.BlockSpec((_ROWS // 5, _COLS), lambda i: (i, 0))],
        out_specs=pl.BlockSpec((_ROWS // 5, _COLS), lambda i: (i, 0)),
    )(x)
    return out.reshape(_N)
